# Initial kernel scaffold; baseline (speedup 1.0000x reference)
import jax
import jax.numpy as jnp
from jax.experimental import pallas as pl


def kernel(context, target, negatives, input_table, output_table):
    raise NotImplementedError("write your pallas kernel here")


# speedup vs baseline: 4.7105x; 4.7105x over previous
import functools

import jax
import jax.numpy as jnp
from jax import lax
from jax.experimental import pallas as pl
from jax.experimental.pallas import tpu as pltpu
from jax.experimental.pallas import tpu_sc as plsc

B = 16384
W = 10
K = 20
D = 64
PAD = 32

NC = 2
NS = 16
NW = NC * NS
NB = B // NW
CB = 32
NCH = NB // CB


def _sc_logits_kernel(ctx_hbm, tgt_hbm, neg_hbm, itab_hbm, otab_hbm,
                      out_hbm, ctx_idx_v, tgt_idx_v, neg_idx_v,
                      ctx_rows_v, tgt_rows_v, neg_rows_v, part_v, logits_v,
                      sem):
  wid = lax.axis_index("s") * NC + lax.axis_index("c")
  lane = lax.iota(jnp.int32, 16)

  def chunk_body(c, carry):
    rb = wid * NB + c * CB
    pltpu.sync_copy(ctx_hbm.at[pl.ds(rb * W, CB * W)], ctx_idx_v)
    pltpu.sync_copy(tgt_hbm.at[pl.ds(rb, CB)], tgt_idx_v)
    pltpu.sync_copy(neg_hbm.at[pl.ds(rb * K, CB * K)], neg_idx_v)
    cp1 = pltpu.async_copy(itab_hbm.at[ctx_idx_v], ctx_rows_v, sem)
    cp2 = pltpu.async_copy(otab_hbm.at[tgt_idx_v], tgt_rows_v, sem)
    cp3 = pltpu.async_copy(otab_hbm.at[neg_idx_v], neg_rows_v, sem)
    cp1.wait()
    cp2.wait()
    cp3.wait()

    def elem_body(b, carry2):
      mean = []
      for v in range(D // 16):
        acc = ctx_rows_v[b * W, pl.ds(v * 16, 16)]
        for w in range(1, W):
          acc = acc + ctx_rows_v[b * W + w, pl.ds(v * 16, 16)]
        mean.append(acc * jnp.float32(1.0 / W))

      def dot_partial(rows_ref, r, k):
        p = mean[0] * rows_ref[r, pl.ds(0, 16)]
        for v in range(1, D // 16):
          p = p + mean[v] * rows_ref[r, pl.ds(v * 16, 16)]
        idx = lane * PAD + k
        plsc.store_scatter(part_v, [idx], p)

      dot_partial(tgt_rows_v, b, 0)
      for k in range(K):
        dot_partial(neg_rows_v, b * K + k, k + 1)
      acc1 = part_v[pl.ds(0, 16)]
      acc2 = part_v[pl.ds(16, 16)]
      for i in range(1, 16):
        acc1 = acc1 + part_v[pl.ds(i * PAD, 16)]
        acc2 = acc2 + part_v[pl.ds(i * PAD + 16, 16)]
      logits_v[b, pl.ds(0, 16)] = acc1
      logits_v[b, pl.ds(16, 16)] = acc2
      return carry2

    lax.fori_loop(0, CB, elem_body, 0)
    pltpu.sync_copy(logits_v, out_hbm.at[pl.ds(rb, CB)])
    return carry

  lax.fori_loop(0, NCH, chunk_body, 0)


@jax.jit
def _sc_logits(ctx_flat, tgt_flat, neg_flat, itab, otab):
  mesh = plsc.VectorSubcoreMesh(core_axis_name="c", subcore_axis_name="s")
  return pl.kernel(
      _sc_logits_kernel,
      mesh=mesh,
      out_type=jax.ShapeDtypeStruct((B, PAD), jnp.float32),
      compiler_params=pltpu.CompilerParams(
          needs_layout_passes=False, use_tc_tiling_on_sc=False),
      scratch_types=[
          pltpu.VMEM((CB * W,), jnp.int32),
          pltpu.VMEM((CB,), jnp.int32),
          pltpu.VMEM((CB * K,), jnp.int32),
          pltpu.VMEM((CB * W, D), jnp.float32),
          pltpu.VMEM((CB, D), jnp.float32),
          pltpu.VMEM((CB * K, D), jnp.float32),
          pltpu.VMEM((16 * PAD,), jnp.float32),
          pltpu.VMEM((CB, PAD), jnp.float32),
          pltpu.SemaphoreType.DMA,
      ],
  )(ctx_flat, tgt_flat, neg_flat, itab, otab)


def _bce_body(l_ref, o_ref):
  x = l_ref[...]
  col = lax.broadcasted_iota(jnp.int32, x.shape, 1)
  label = (col == 0).astype(x.dtype)
  loss = jnp.maximum(x, 0.0) - x * label + jnp.log1p(jnp.exp(-jnp.abs(x)))
  loss = jnp.where(col < (K + 1), loss, 0.0)
  o_ref[0, 0] = jnp.sum(loss) / jnp.float32(B * (K + 1))


@jax.jit
def _bce_mean(logits):
  out = pl.pallas_call(
      _bce_body,
      out_shape=jax.ShapeDtypeStruct((1, 1), jnp.float32),
      in_specs=[pl.BlockSpec(memory_space=pltpu.VMEM)],
      out_specs=pl.BlockSpec(memory_space=pltpu.SMEM),
  )(logits)
  return out[0, 0]


def kernel(context, target, negatives, input_table, output_table):
  ctx_flat = context.astype(jnp.int32).reshape(-1)
  tgt_flat = target.astype(jnp.int32).reshape(-1)
  neg_flat = negatives.astype(jnp.int32).reshape(-1)
  logits = _sc_logits(ctx_flat, tgt_flat, neg_flat, input_table, output_table)
  return _bce_mean(logits)


# speedup vs baseline: 4.9582x; 1.0526x over previous
import jax
import jax.numpy as jnp
from jax import lax
from jax.experimental import pallas as pl
from jax.experimental.pallas import tpu as pltpu
from jax.experimental.pallas import tpu_sc as plsc

B = 16384
W = 10
K = 20
D = 64
PAD = 32

NC = 2
NS = 16
NW = NC * NS
NB = B // NW
CB = 16
NCH = NB // CB


def _sc_logits_kernel(ctx_hbm, tgt_hbm, neg_hbm, itab_hbm, otab_hbm,
                      out_hbm,
                      ctx_idx_a, tgt_idx_a, neg_idx_a,
                      ctx_rows_a, tgt_rows_a, neg_rows_a,
                      ctx_idx_b, tgt_idx_b, neg_idx_b,
                      ctx_rows_b, tgt_rows_b, neg_rows_b,
                      part_v, logits_v, sem_ga, sem_gb, sem_ia, sem_ib):
  wid = lax.axis_index("s") * NC + lax.axis_index("c")
  lane = lax.iota(jnp.int32, 16)
  bufs = (
      (ctx_idx_a, tgt_idx_a, neg_idx_a, ctx_rows_a, tgt_rows_a, neg_rows_a,
       sem_ga, sem_ia),
      (ctx_idx_b, tgt_idx_b, neg_idx_b, ctx_rows_b, tgt_rows_b, neg_rows_b,
       sem_gb, sem_ib),
  )

  def i_copies(c, s):
    rb = wid * NB + c * CB
    ci, ti, ni = bufs[s][0], bufs[s][1], bufs[s][2]
    si = bufs[s][7]
    return (
        pltpu.make_async_copy(ctx_hbm.at[pl.ds(rb * W, CB * W)], ci, si),
        pltpu.make_async_copy(tgt_hbm.at[pl.ds(rb, CB)], ti, si),
        pltpu.make_async_copy(neg_hbm.at[pl.ds(rb * K, CB * K)], ni, si),
    )

  def g_copies(s):
    ci, ti, ni, cr, tr, nr, sg = bufs[s][:7]
    return (
        pltpu.make_async_copy(itab_hbm.at[ci], cr, sg),
        pltpu.make_async_copy(otab_hbm.at[ti], tr, sg),
        pltpu.make_async_copy(otab_hbm.at[ni], nr, sg),
    )

  def issue(cps):
    for cp in cps:
      cp.start()

  def drain(cps):
    for cp in cps:
      cp.wait()

  def compute(s, c):
    ctx_rows_v, tgt_rows_v, neg_rows_v = bufs[s][3], bufs[s][4], bufs[s][5]

    def elem_body(b, carry2):
      mean = []
      for v in range(D // 16):
        acc = ctx_rows_v[b * W, pl.ds(v * 16, 16)]
        for w in range(1, W):
          acc = acc + ctx_rows_v[b * W + w, pl.ds(v * 16, 16)]
        mean.append(acc * jnp.float32(1.0 / W))

      def dot_partial(rows_ref, r, k):
        p = mean[0] * rows_ref[r, pl.ds(0, 16)]
        for v in range(1, D // 16):
          p = p + mean[v] * rows_ref[r, pl.ds(v * 16, 16)]
        idx = lane * PAD + k
        plsc.store_scatter(part_v, [idx], p)

      dot_partial(tgt_rows_v, b, 0)
      for k in range(K):
        dot_partial(neg_rows_v, b * K + k, k + 1)
      acc1 = part_v[pl.ds(0, 16)]
      acc2 = part_v[pl.ds(16, 16)]
      for i in range(1, 16):
        acc1 = acc1 + part_v[pl.ds(i * PAD, 16)]
        acc2 = acc2 + part_v[pl.ds(i * PAD + 16, 16)]
      logits_v[b, pl.ds(0, 16)] = acc1
      logits_v[b, pl.ds(16, 16)] = acc2
      return carry2

    lax.fori_loop(0, CB, elem_body, 0)
    rb = wid * NB + c * CB
    pltpu.sync_copy(logits_v, out_hbm.at[pl.ds(rb, CB)])

  issue(i_copies(0, 0))
  drain(i_copies(0, 0))
  issue(g_copies(0))
  issue(i_copies(1, 1))
  T = NCH // 2

  def body(t, carry):
    c0 = 2 * t
    drain(g_copies(0))
    drain(i_copies(c0 + 1, 1))
    issue(g_copies(1))

    @pl.when(t < T - 1)
    def _():
      issue(i_copies(c0 + 2, 0))

    compute(0, c0)
    drain(g_copies(1))

    @pl.when(t < T - 1)
    def _():
      drain(i_copies(c0 + 2, 0))
      issue(g_copies(0))
      issue(i_copies(c0 + 3, 1))

    compute(1, c0 + 1)
    return carry

  lax.fori_loop(0, T, body, 0)


@jax.jit
def _sc_logits(ctx_flat, tgt_flat, neg_flat, itab, otab):
  mesh = plsc.VectorSubcoreMesh(core_axis_name="c", subcore_axis_name="s")
  return pl.kernel(
      _sc_logits_kernel,
      mesh=mesh,
      out_type=jax.ShapeDtypeStruct((B, PAD), jnp.float32),
      compiler_params=pltpu.CompilerParams(
          needs_layout_passes=False, use_tc_tiling_on_sc=False),
      scratch_types=[
          pltpu.VMEM((CB * W,), jnp.int32),
          pltpu.VMEM((CB,), jnp.int32),
          pltpu.VMEM((CB * K,), jnp.int32),
          pltpu.VMEM((CB * W, D), jnp.float32),
          pltpu.VMEM((CB, D), jnp.float32),
          pltpu.VMEM((CB * K, D), jnp.float32),
          pltpu.VMEM((CB * W,), jnp.int32),
          pltpu.VMEM((CB,), jnp.int32),
          pltpu.VMEM((CB * K,), jnp.int32),
          pltpu.VMEM((CB * W, D), jnp.float32),
          pltpu.VMEM((CB, D), jnp.float32),
          pltpu.VMEM((CB * K, D), jnp.float32),
          pltpu.VMEM((16 * PAD,), jnp.float32),
          pltpu.VMEM((CB, PAD), jnp.float32),
          pltpu.SemaphoreType.DMA,
          pltpu.SemaphoreType.DMA,
          pltpu.SemaphoreType.DMA,
          pltpu.SemaphoreType.DMA,
      ],
  )(ctx_flat, tgt_flat, neg_flat, itab, otab)


def _bce_body(l_ref, o_ref):
  x = l_ref[...]
  col = lax.broadcasted_iota(jnp.int32, x.shape, 1)
  label = (col == 0).astype(x.dtype)
  loss = jnp.maximum(x, 0.0) - x * label + jnp.log1p(jnp.exp(-jnp.abs(x)))
  loss = jnp.where(col < (K + 1), loss, 0.0)
  o_ref[0, 0] = jnp.sum(loss) / jnp.float32(B * (K + 1))


@jax.jit
def _bce_mean(logits):
  out = pl.pallas_call(
      _bce_body,
      out_shape=jax.ShapeDtypeStruct((1, 1), jnp.float32),
      in_specs=[pl.BlockSpec(memory_space=pltpu.VMEM)],
      out_specs=pl.BlockSpec(memory_space=pltpu.SMEM),
  )(logits)
  return out[0, 0]


def kernel(context, target, negatives, input_table, output_table):
  ctx_flat = context.astype(jnp.int32).reshape(-1)
  tgt_flat = target.astype(jnp.int32).reshape(-1)
  neg_flat = negatives.astype(jnp.int32).reshape(-1)
  logits = _sc_logits(ctx_flat, tgt_flat, neg_flat, input_table, output_table)
  return _bce_mean(logits)
